# baseline (speedup 1.0000x reference)
import functools

import jax
import jax.numpy as jnp
from jax import lax
from jax.experimental import pallas as pl
from jax.experimental.pallas import tpu as pltpu
from jax.experimental.pallas import tpu_sc as plsc

_BATCH = 16384
_HIDDEN = 64
_ROWS = 100001
_SPLIT = 50176
_RPK_BLK = 512
_RPK_STEPS = _SPLIT // _RPK_BLK
_NUM_CORES = 2
_NUM_SUBCORES = 16
_NW = _NUM_CORES * _NUM_SUBCORES
_BPW = _BATCH // _NW


def _repack_body(ua_ref, ub_ref, ia_ref, ib_ref, uo_ref, io_ref):
    uo_ref[...] = jnp.concatenate([ua_ref[...], ub_ref[...]], axis=1)
    io_ref[...] = jnp.concatenate([ia_ref[...], ib_ref[...]], axis=1)


_tc_repack = pl.pallas_call(
    _repack_body,
    grid=(_RPK_STEPS,),
    in_specs=[
        pl.BlockSpec((_RPK_BLK, _HIDDEN), lambda i: (i, 0)),
        pl.BlockSpec((_RPK_BLK, _HIDDEN), lambda i: (_RPK_STEPS + i, 0)),
        pl.BlockSpec((_RPK_BLK, _HIDDEN), lambda i: (i, 0)),
        pl.BlockSpec((_RPK_BLK, _HIDDEN), lambda i: (_RPK_STEPS + i, 0)),
    ],
    out_specs=[
        pl.BlockSpec((_RPK_BLK, 2 * _HIDDEN), lambda i: (i, 0)),
        pl.BlockSpec((_RPK_BLK, 2 * _HIDDEN), lambda i: (i, 0)),
    ],
    out_shape=(
        jax.ShapeDtypeStruct((_SPLIT, 2 * _HIDDEN), jnp.float32),
        jax.ShapeDtypeStruct((_SPLIT, 2 * _HIDDEN), jnp.float32),
    ),
)


_sc_mesh = plsc.VectorSubcoreMesh(core_axis_name="c", subcore_axis_name="s")


@functools.partial(
    pl.kernel,
    out_type=(
        jax.ShapeDtypeStruct((_BATCH, 2 * _HIDDEN), jnp.float32),
        jax.ShapeDtypeStruct((_BATCH, 2 * _HIDDEN), jnp.float32),
    ),
    mesh=_sc_mesh,
    scratch_types=[
        pltpu.VMEM((_BPW,), jnp.int32),
        pltpu.VMEM((_BPW,), jnp.int32),
        pltpu.VMEM((_BPW, 2 * _HIDDEN), jnp.float32),
        pltpu.SemaphoreType.DMA,
    ],
)
def _sc_gather(uj_hbm, ij_hbm, utab_hbm, itab_hbm, uout_hbm, iout_hbm,
               uidx_v, iidx_v, rows_v, sem):
    wid = lax.axis_index("s") * _NUM_CORES + lax.axis_index("c")
    base = wid * _BPW
    pltpu.sync_copy(uj_hbm.at[pl.ds(base, _BPW)], uidx_v)
    pltpu.sync_copy(ij_hbm.at[pl.ds(base, _BPW)], iidx_v)
    pltpu.async_copy(utab_hbm.at[uidx_v], rows_v, sem).wait()
    pltpu.sync_copy(rows_v, uout_hbm.at[pl.ds(base, _BPW)])
    pltpu.async_copy(itab_hbm.at[iidx_v], rows_v, sem).wait()
    pltpu.sync_copy(rows_v, iout_hbm.at[pl.ds(base, _BPW)])


_NRM_BLK = 2048


def _norm_body(uraw_ref, iraw_ref, uk_ref, ik_ref, uo_ref, io_ref):
    for raw_ref, k_ref, o_ref in ((uraw_ref, uk_ref, uo_ref),
                                  (iraw_ref, ik_ref, io_ref)):
        raw = raw_ref[...]
        sel = jnp.where(k_ref[...] > 0.5, raw[:, _HIDDEN:], raw[:, :_HIDDEN])
        norm = jnp.sqrt(jnp.sum(sel * sel, axis=1, keepdims=True))
        o_ref[...] = sel / jnp.maximum(norm, 1e-12)


_tc_normalize = pl.pallas_call(
    _norm_body,
    grid=(_BATCH // _NRM_BLK,),
    in_specs=[
        pl.BlockSpec((_NRM_BLK, 2 * _HIDDEN), lambda i: (i, 0)),
        pl.BlockSpec((_NRM_BLK, 2 * _HIDDEN), lambda i: (i, 0)),
        pl.BlockSpec((_NRM_BLK, 1), lambda i: (i, 0)),
        pl.BlockSpec((_NRM_BLK, 1), lambda i: (i, 0)),
    ],
    out_specs=[
        pl.BlockSpec((_NRM_BLK, _HIDDEN), lambda i: (i, 0)),
        pl.BlockSpec((_NRM_BLK, _HIDDEN), lambda i: (i, 0)),
    ],
    out_shape=(
        jax.ShapeDtypeStruct((_BATCH, _HIDDEN), jnp.float32),
        jax.ShapeDtypeStruct((_BATCH, _HIDDEN), jnp.float32),
    ),
)


def kernel(user_ids, item_ids, user_table, item_table):
    uid = user_ids.astype(jnp.int32)
    iid = item_ids.astype(jnp.int32)
    uj = jnp.where(uid < _SPLIT, uid, uid - _SPLIT)
    ij = jnp.where(iid < _SPLIT, iid, iid - _SPLIT)
    uk = (uid >= _SPLIT).astype(jnp.float32).reshape(_BATCH, 1)
    ik = (iid >= _SPLIT).astype(jnp.float32).reshape(_BATCH, 1)
    utab, itab = _tc_repack(user_table, user_table, item_table, item_table)
    uraw, iraw = _sc_gather(uj, ij, utab, itab)
    return _tc_normalize(uraw, iraw, uk, ik)
